# baseline (speedup 1.0000x reference)
import jax
import jax.numpy as jnp
from jax import lax
from jax.experimental import pallas as pl
from jax.experimental.pallas import tpu as pltpu

N_EXPERTS = 64
D_MODEL = 4096
SEQ = 8192
BATCH = 4
CHUNK = 512
CHUNKS_PER_B = SEQ // CHUNK
N_CHUNKS = BATCH * CHUNKS_PER_B
NBUF = 4


def _finalize(acc, w_ref, ph_ref, tk_ref, ts_ref, ti_ref, coh_ref, b):
    pooled = jnp.sum(acc, axis=0, keepdims=True) * (1.0 / SEQ)
    amp = lax.dot_general(
        pooled, w_ref[...], (((1,), (1,)), ((), ())),
        preferred_element_type=jnp.float32,
    )
    ph = ph_ref[...]
    coh = jnp.abs(amp * (jnp.cos(ph) + jnp.sin(ph)))
    coh_ref[pl.ds(b, 1), :] = coh

    iota = lax.broadcasted_iota(jnp.int32, (1, N_EXPERTS), 1)
    m1 = jnp.max(coh, axis=1, keepdims=True)
    i1 = jnp.min(jnp.where(coh == m1, iota, N_EXPERTS),
                 axis=1, keepdims=True)
    coh2 = jnp.where(iota == i1, -1.0, coh)
    m2 = jnp.max(coh2, axis=1, keepdims=True)
    i2 = jnp.min(jnp.where(coh2 == m2, iota, N_EXPERTS),
                 axis=1, keepdims=True)
    delta = (tk_ref[0, 0] - 2).astype(jnp.float32)
    ts_ref[pl.ds(b, 1), :] = jnp.concatenate([m1, m2], axis=1) + delta
    ti_ref[pl.ds(b, 1), :] = jnp.concatenate([i1, i2], axis=1)


def _router_body(x_ref, w_ref, ph_ref, tk_ref, ts_ref, ti_ref, coh_ref,
                 b0, b1, b2, b3, s0, s1, s2, s3):
    bufs = (b0, b1, b2, b3)
    sems = (s0, s1, s2, s3)

    def start(g):
        b, ci = divmod(g, CHUNKS_PER_B)
        return pltpu.make_async_copy(
            x_ref.at[b, pl.ds(ci * CHUNK, CHUNK), :],
            bufs[g % NBUF], sems[g % NBUF])

    for g in range(NBUF):
        start(g).start()

    acc = None
    for g in range(N_CHUNKS):
        start(g).wait()
        s = jnp.sum(
            bufs[g % NBUF][...].reshape(CHUNK // 8, 8, D_MODEL), axis=0)
        acc = s if acc is None else acc + s
        if g + NBUF < N_CHUNKS:
            start(g + NBUF).start()
        if (g + 1) % CHUNKS_PER_B == 0:
            _finalize(acc, w_ref, ph_ref, tk_ref, ts_ref, ti_ref, coh_ref,
                      g // CHUNKS_PER_B)
            acc = None


def kernel(x, W, phase_angles, top_k):
    ph2 = phase_angles.reshape(1, N_EXPERTS)
    tk = jnp.asarray(top_k, jnp.int32).reshape(1, 1)
    return pl.pallas_call(
        _router_body,
        in_specs=[
            pl.BlockSpec(memory_space=pl.ANY),
            pl.BlockSpec((N_EXPERTS, D_MODEL), lambda: (0, 0)),
            pl.BlockSpec((1, N_EXPERTS), lambda: (0, 0)),
            pl.BlockSpec(memory_space=pltpu.SMEM),
        ],
        out_specs=[
            pl.BlockSpec((BATCH, 2), lambda: (0, 0)),
            pl.BlockSpec((BATCH, 2), lambda: (0, 0)),
            pl.BlockSpec((BATCH, N_EXPERTS), lambda: (0, 0)),
        ],
        out_shape=[
            jax.ShapeDtypeStruct((BATCH, 2), jnp.float32),
            jax.ShapeDtypeStruct((BATCH, 2), jnp.int32),
            jax.ShapeDtypeStruct((BATCH, N_EXPERTS), jnp.float32),
        ],
        scratch_shapes=(
            [pltpu.VMEM((CHUNK, D_MODEL), jnp.float32)] * NBUF
            + [pltpu.SemaphoreType.DMA] * NBUF
        ),
    )(x, W, ph2, tk)


# speedup vs baseline: 1.0246x; 1.0246x over previous
import jax
import jax.numpy as jnp
from jax import lax
from jax.experimental import pallas as pl
from jax.experimental.pallas import tpu as pltpu

N_EXPERTS = 64
D_MODEL = 4096
SEQ = 8192
BATCH = 4
CHUNK = 1024
N_CHUNKS = SEQ // CHUNK


def _router_body(x_ref, w_ref, ph_ref, tk_ref, ts_ref, ti_ref, coh_ref,
                 acc_ref):
    b = pl.program_id(0)
    c = pl.program_id(1)

    @pl.when(c == 0)
    def _init():
        acc_ref[...] = jnp.zeros_like(acc_ref)

    acc_ref[...] += jnp.sum(
        x_ref[...].reshape(CHUNK // 8, 8, D_MODEL), axis=0)

    @pl.when(c == N_CHUNKS - 1)
    def _finalize():
        pooled = jnp.sum(acc_ref[...], axis=0, keepdims=True) * (1.0 / SEQ)
        amp = lax.dot_general(
            pooled, w_ref[...], (((1,), (1,)), ((), ())),
            preferred_element_type=jnp.float32,
        )
        ph = ph_ref[...]
        coh = jnp.abs(amp * (jnp.cos(ph) + jnp.sin(ph)))
        coh_ref[pl.ds(b, 1), :] = coh

        iota = lax.broadcasted_iota(jnp.int32, (1, N_EXPERTS), 1)
        m1 = jnp.max(coh, axis=1, keepdims=True)
        i1 = jnp.min(jnp.where(coh == m1, iota, N_EXPERTS),
                     axis=1, keepdims=True)
        coh2 = jnp.where(iota == i1, -1.0, coh)
        m2 = jnp.max(coh2, axis=1, keepdims=True)
        i2 = jnp.min(jnp.where(coh2 == m2, iota, N_EXPERTS),
                     axis=1, keepdims=True)
        delta = (tk_ref[0, 0] - 2).astype(jnp.float32)
        ts_ref[pl.ds(b, 1), :] = jnp.concatenate([m1, m2], axis=1) + delta
        ti_ref[pl.ds(b, 1), :] = jnp.concatenate([i1, i2], axis=1)


def kernel(x, W, phase_angles, top_k):
    ph2 = phase_angles.reshape(1, N_EXPERTS)
    tk = jnp.asarray(top_k, jnp.int32).reshape(1, 1)
    x2 = x.reshape(BATCH * SEQ, D_MODEL)
    return pl.pallas_call(
        _router_body,
        grid=(BATCH, N_CHUNKS),
        in_specs=[
            pl.BlockSpec((CHUNK, D_MODEL),
                         lambda b, c: (b * N_CHUNKS + c, 0)),
            pl.BlockSpec((N_EXPERTS, D_MODEL), lambda b, c: (0, 0)),
            pl.BlockSpec((1, N_EXPERTS), lambda b, c: (0, 0)),
            pl.BlockSpec(memory_space=pltpu.SMEM),
        ],
        out_specs=[
            pl.BlockSpec((BATCH, 2), lambda b, c: (0, 0)),
            pl.BlockSpec((BATCH, 2), lambda b, c: (0, 0)),
            pl.BlockSpec((BATCH, N_EXPERTS), lambda b, c: (0, 0)),
        ],
        out_shape=[
            jax.ShapeDtypeStruct((BATCH, 2), jnp.float32),
            jax.ShapeDtypeStruct((BATCH, 2), jnp.int32),
            jax.ShapeDtypeStruct((BATCH, N_EXPERTS), jnp.float32),
        ],
        scratch_shapes=[pltpu.VMEM((8, D_MODEL), jnp.float32)],
        compiler_params=pltpu.CompilerParams(
            dimension_semantics=("arbitrary", "arbitrary"),
        ),
    )(x2, W, ph2, tk)
